# baseline (speedup 1.0000x reference)
import jax
import jax.numpy as jnp
from jax import lax
from jax.experimental import pallas as pl
from jax.experimental.pallas import tpu as pltpu
from jax.experimental.pallas import tpu_sc as plsc

_N, _H, _W = 32, 256, 256
_HW = _H * _W
_P = 8192
_CLS_W, _REG_W = 1.0, 1.2

_NC, _NS, _L = 2, 16, 16
_NWORK = _NC * _NS
_BPW = _P // _NWORK
_NCHUNK = _BPW // _L

_GRID = 8
_ROWS = _N * _H
_BLKR = _ROWS // _GRID


def _sc_body(pbi_hbm, pmi_hbm, boxes_hbm, gt_hbm, score_hbm, part_hbm,
             pbi_v, pmi_v, key_v, key4_v, pcol_v, scr_v, gt_v, acc_v, sem):
    wid = lax.axis_index("s") * _NC + lax.axis_index("c")
    base = wid * _BPW
    pltpu.sync_copy(pbi_hbm.at[pl.ds(base, _BPW)], pbi_v)
    pltpu.sync_copy(pmi_hbm.at[pl.ds(base, _BPW)], pmi_v)
    pltpu.sync_copy(gt_hbm, gt_v)

    for j in range(_NCHUNK):
        pb = pbi_v[pl.ds(j * _L, _L)]
        pm = pmi_v[pl.ds(j * _L, _L)]
        skey = (pb * _HW + ((pm >> 11) << 11) + ((pm >> 7) & 1) * 1024
                + (((pm >> 8) & 7) << 7) + (pm & 127))
        key_v[j // 8, pl.ds((j % 8) * _L, _L)] = skey
        base = pb * (_HW * 4) + ((pm >> 7) << 9) + (pm & 127)
        for c in range(4):
            key4_v[c, j // 8, pl.ds((j % 8) * _L, _L)] = base + (c << 7)

    descs = []
    for half in range(2):
        descs.append(pltpu.async_copy(score_hbm.at[key_v.at[half]],
                                      scr_v.at[half], sem))
        for c in range(4):
            descs.append(pltpu.async_copy(boxes_hbm.at[key4_v.at[c, half]],
                                          pcol_v.at[c, half], sem))
    for d in descs:
        d.wait()

    accd = jnp.zeros((_L,), jnp.float32)
    accr = jnp.zeros((_L,), jnp.float32)
    cols = [jnp.full((_L,), c, jnp.int32) for c in range(4)]
    for j in range(_NCHUNK):
        pb = pbi_v[pl.ds(j * _L, _L)]
        px1 = pcol_v[0, j // 8, pl.ds((j % 8) * _L, _L)]
        py1 = pcol_v[1, j // 8, pl.ds((j % 8) * _L, _L)]
        px2 = pcol_v[2, j // 8, pl.ds((j % 8) * _L, _L)]
        py2 = pcol_v[3, j // 8, pl.ds((j % 8) * _L, _L)]
        gx1 = plsc.load_gather(gt_v, [pb, cols[0]])
        gy1 = plsc.load_gather(gt_v, [pb, cols[1]])
        gx2 = plsc.load_gather(gt_v, [pb, cols[2]])
        gy2 = plsc.load_gather(gt_v, [pb, cols[3]])
        ww = jnp.maximum(jnp.minimum(gx2, px2) - jnp.maximum(gx1, px1), 0.0)
        hh = jnp.maximum(jnp.minimum(gy2, py2) - jnp.maximum(gy1, py1), 0.0)
        overlap = ww * hh
        a1 = (gx2 - gx1) * (gy2 - gy1)
        a2 = (px2 - px1) * (py2 - py1)
        union = jnp.maximum(a1 + a2 - overlap, 1e-6)
        iou = overlap / union
        sc = scr_v[j // 8, pl.ds((j % 8) * _L, _L)]
        accd = accd + sc * iou
        accr = accr + (jnp.abs(px1 - gx1) + jnp.abs(py1 - gy1)
                       + jnp.abs(px2 - gx2) + jnp.abs(py2 - gy2))

    acc_v[0] = accd
    acc_v[1] = accr
    pltpu.sync_copy(acc_v.at[0], part_hbm.at[wid])
    pltpu.sync_copy(acc_v.at[1], part_hbm.at[wid + _NWORK])


def _make_sc_call():
    return pl.kernel(
        _sc_body,
        out_type=jax.ShapeDtypeStruct((2 * _NWORK, _L), jnp.float32),
        mesh=plsc.VectorSubcoreMesh(core_axis_name="c", subcore_axis_name="s",
                                    num_cores=_NC, num_subcores=_NS),
        compiler_params=pltpu.CompilerParams(needs_layout_passes=False),
        scratch_types=[
            pltpu.VMEM((_BPW,), jnp.int32),
            pltpu.VMEM((_BPW,), jnp.int32),
            pltpu.VMEM((2, 128), jnp.int32),
            pltpu.VMEM((4, 2, 128), jnp.int32),
            pltpu.VMEM((4, 2, 128), jnp.float32),
            pltpu.VMEM((2, 128), jnp.float32),
            pltpu.VMEM((_N, 4), jnp.float32),
            pltpu.VMEM((2, _L), jnp.float32),
            pltpu.SemaphoreType.DMA,
        ],
    )


def _tc_body(np_ref, score_ref, part_ref, tot_ref, cls_ref, reg_ref, acc_ref):
    i = pl.program_id(0)

    @pl.when(i == 0)
    def _init():
        acc_ref[0, 0] = 0.0

    x = score_ref[...]
    f = jnp.maximum(x, 0.0) + jnp.log(1.0 + jnp.exp(-jnp.abs(x)))
    acc_ref[0, 0] += jnp.sum(f)

    @pl.when(i == _GRID - 1)
    def _fin():
        p = part_ref[...]
        dot = jnp.sum(p[:_NWORK, :])
        regs = jnp.sum(p[_NWORK:, :])
        npos = jnp.maximum(np_ref[0, 0], 1.0)
        cls = (acc_ref[0, 0] - dot) / npos * _CLS_W
        reg = regs / npos * _REG_W
        tot_ref[0, 0] = cls + reg
        cls_ref[0, 0] = cls
        reg_ref[0, 0] = reg


_tc_call = pl.pallas_call(
    _tc_body,
    grid=(_GRID,),
    in_specs=[
        pl.BlockSpec(memory_space=pltpu.SMEM),
        pl.BlockSpec((_BLKR, _W), lambda i: (i, 0)),
        pl.BlockSpec((2 * _NWORK, _L), lambda i: (0, 0)),
    ],
    out_specs=[pl.BlockSpec(memory_space=pltpu.SMEM)] * 3,
    out_shape=[jax.ShapeDtypeStruct((1, 1), jnp.float32)] * 3,
    scratch_shapes=[pltpu.SMEM((1, 1), jnp.float32)],
)


def kernel(score_map, boxes, gt_boxes, num_positive_samples,
           positive_batch_idx, positive_map_idx):
    score_flat = (score_map.astype(jnp.float32)
                  .reshape(_N, _H // 8, 8, _W // 128, 128)
                  .transpose(0, 1, 3, 2, 4)
                  .reshape(-1))
    boxes_flat = (boxes.astype(jnp.float32)
                  .reshape(_N, _HW // 128, 128, 4)
                  .transpose(0, 1, 3, 2)
                  .reshape(-1))
    gt = gt_boxes.astype(jnp.float32)
    part = _make_sc_call()(positive_batch_idx, positive_map_idx, boxes_flat,
                           gt, score_flat)
    score2d = score_map.astype(jnp.float32).reshape(_ROWS, _W)
    np11 = jnp.reshape(num_positive_samples, (1, 1)).astype(jnp.float32)
    tot, cls, reg = _tc_call(np11, score2d, part)
    return (jnp.reshape(tot, ()), jnp.reshape(cls, ()), jnp.reshape(reg, ()))


# speedup vs baseline: 1.1704x; 1.1704x over previous
import jax
import jax.numpy as jnp
from jax import lax
from jax.experimental import pallas as pl
from jax.experimental.pallas import tpu as pltpu
from jax.experimental.pallas import tpu_sc as plsc

_N, _H, _W = 32, 256, 256
_HW = _H * _W
_P = 8192
_CLS_W, _REG_W = 1.0, 1.2

_NC, _NS, _L = 2, 16, 16
_NWORK = _NC * _NS
_BPW = _P // _NWORK
_NCHUNK = _BPW // _L

_GRID = 4
_ROWS = _N * _H
_BLKR = _ROWS // _GRID


def _sc_body(pbi_hbm, pmi_hbm, boxes_hbm, gt_hbm, score_hbm, part_hbm,
             pbi_v, pmi_v, key_v, key4_v, pcol_v, scr_v, gt_v, acc_v, sem):
    wid = lax.axis_index("s") * _NC + lax.axis_index("c")
    base = wid * _BPW
    pltpu.sync_copy(pbi_hbm.at[pl.ds(base, _BPW)], pbi_v)
    pltpu.sync_copy(pmi_hbm.at[pl.ds(base, _BPW)], pmi_v)
    pltpu.sync_copy(gt_hbm, gt_v)

    for j in range(_NCHUNK):
        pb = pbi_v[pl.ds(j * _L, _L)]
        pm = pmi_v[pl.ds(j * _L, _L)]
        skey = (pb * _HW + ((pm >> 11) << 11) + ((pm >> 7) & 1) * 1024
                + (((pm >> 8) & 7) << 7) + (pm & 127))
        key_v[j // 8, pl.ds((j % 8) * _L, _L)] = skey
        base = pb * (_HW * 4) + ((pm >> 7) << 9) + (pm & 127)
        for c in range(4):
            key4_v[c, j // 8, pl.ds((j % 8) * _L, _L)] = base + (c << 7)

    descs = []
    for half in range(2):
        descs.append(pltpu.async_copy(score_hbm.at[key_v.at[half]],
                                      scr_v.at[half], sem))
        for c in range(4):
            descs.append(pltpu.async_copy(boxes_hbm.at[key4_v.at[c, half]],
                                          pcol_v.at[c, half], sem))
    for d in descs:
        d.wait()

    accd = jnp.zeros((_L,), jnp.float32)
    accr = jnp.zeros((_L,), jnp.float32)
    cols = [jnp.full((_L,), c, jnp.int32) for c in range(4)]
    for j in range(_NCHUNK):
        pb = pbi_v[pl.ds(j * _L, _L)]
        px1 = pcol_v[0, j // 8, pl.ds((j % 8) * _L, _L)]
        py1 = pcol_v[1, j // 8, pl.ds((j % 8) * _L, _L)]
        px2 = pcol_v[2, j // 8, pl.ds((j % 8) * _L, _L)]
        py2 = pcol_v[3, j // 8, pl.ds((j % 8) * _L, _L)]
        gx1 = plsc.load_gather(gt_v, [pb, cols[0]])
        gy1 = plsc.load_gather(gt_v, [pb, cols[1]])
        gx2 = plsc.load_gather(gt_v, [pb, cols[2]])
        gy2 = plsc.load_gather(gt_v, [pb, cols[3]])
        ww = jnp.maximum(jnp.minimum(gx2, px2) - jnp.maximum(gx1, px1), 0.0)
        hh = jnp.maximum(jnp.minimum(gy2, py2) - jnp.maximum(gy1, py1), 0.0)
        overlap = ww * hh
        a1 = (gx2 - gx1) * (gy2 - gy1)
        a2 = (px2 - px1) * (py2 - py1)
        union = jnp.maximum(a1 + a2 - overlap, 1e-6)
        iou = overlap / union
        sc = scr_v[j // 8, pl.ds((j % 8) * _L, _L)]
        accd = accd + sc * iou
        accr = accr + (jnp.abs(px1 - gx1) + jnp.abs(py1 - gy1)
                       + jnp.abs(px2 - gx2) + jnp.abs(py2 - gy2))

    acc_v[0] = accd
    acc_v[1] = accr
    pltpu.sync_copy(acc_v.at[0], part_hbm.at[wid])
    pltpu.sync_copy(acc_v.at[1], part_hbm.at[wid + _NWORK])


def _make_sc_call():
    return pl.kernel(
        _sc_body,
        out_type=jax.ShapeDtypeStruct((2 * _NWORK, _L), jnp.float32),
        mesh=plsc.VectorSubcoreMesh(core_axis_name="c", subcore_axis_name="s",
                                    num_cores=_NC, num_subcores=_NS),
        compiler_params=pltpu.CompilerParams(needs_layout_passes=False),
        scratch_types=[
            pltpu.VMEM((_BPW,), jnp.int32),
            pltpu.VMEM((_BPW,), jnp.int32),
            pltpu.VMEM((2, 128), jnp.int32),
            pltpu.VMEM((4, 2, 128), jnp.int32),
            pltpu.VMEM((4, 2, 128), jnp.float32),
            pltpu.VMEM((2, 128), jnp.float32),
            pltpu.VMEM((_N, 4), jnp.float32),
            pltpu.VMEM((2, _L), jnp.float32),
            pltpu.SemaphoreType.DMA,
        ],
    )


def _tc_body(score_ref, fsum_ref, acc_ref):
    i = pl.program_id(0)

    @pl.when(i == 0)
    def _init():
        acc_ref[0, 0] = 0.0

    x = score_ref[...]
    f = jnp.maximum(x, 0.0) + jnp.log(1.0 + jnp.exp(-jnp.abs(x)))
    acc_ref[0, 0] += jnp.sum(f)

    @pl.when(i == _GRID - 1)
    def _fin():
        fsum_ref[0, 0] = acc_ref[0, 0]


_tc_call = pl.pallas_call(
    _tc_body,
    grid=(_GRID,),
    in_specs=[pl.BlockSpec((_BLKR, _W), lambda i: (i, 0))],
    out_specs=pl.BlockSpec(memory_space=pltpu.SMEM),
    out_shape=jax.ShapeDtypeStruct((1, 1), jnp.float32),
    scratch_shapes=[pltpu.SMEM((1, 1), jnp.float32)],
)


def _comb_body(np_ref, fsum_ref, part_ref, tot_ref, cls_ref, reg_ref):
    p = part_ref[...]
    dot = jnp.sum(p[:_NWORK, :])
    regs = jnp.sum(p[_NWORK:, :])
    npos = jnp.maximum(np_ref[0, 0], 1.0)
    cls = (fsum_ref[0, 0] - dot) / npos * _CLS_W
    reg = regs / npos * _REG_W
    tot_ref[0, 0] = cls + reg
    cls_ref[0, 0] = cls
    reg_ref[0, 0] = reg


_comb_call = pl.pallas_call(
    _comb_body,
    in_specs=[
        pl.BlockSpec(memory_space=pltpu.SMEM),
        pl.BlockSpec(memory_space=pltpu.SMEM),
        pl.BlockSpec((2 * _NWORK, _L), lambda: (0, 0)),
    ],
    out_specs=[pl.BlockSpec(memory_space=pltpu.SMEM)] * 3,
    out_shape=[jax.ShapeDtypeStruct((1, 1), jnp.float32)] * 3,
)


def kernel(score_map, boxes, gt_boxes, num_positive_samples,
           positive_batch_idx, positive_map_idx):
    score_flat = (score_map.astype(jnp.float32)
                  .reshape(_N, _H // 8, 8, _W // 128, 128)
                  .transpose(0, 1, 3, 2, 4)
                  .reshape(-1))
    boxes_flat = (boxes.astype(jnp.float32)
                  .reshape(_N, _HW // 128, 128, 4)
                  .transpose(0, 1, 3, 2)
                  .reshape(-1))
    gt = gt_boxes.astype(jnp.float32)
    part = _make_sc_call()(positive_batch_idx, positive_map_idx, boxes_flat,
                           gt, score_flat)
    score2d = score_map.astype(jnp.float32).reshape(_ROWS, _W)
    np11 = jnp.reshape(num_positive_samples, (1, 1)).astype(jnp.float32)
    fsum = _tc_call(score2d)
    tot, cls, reg = _comb_call(np11, fsum, part)
    return (jnp.reshape(tot, ()), jnp.reshape(cls, ()), jnp.reshape(reg, ()))


# speedup vs baseline: 1.2308x; 1.0516x over previous
import jax
import jax.numpy as jnp
from jax import lax
from jax.experimental import pallas as pl
from jax.experimental.pallas import tpu as pltpu
from jax.experimental.pallas import tpu_sc as plsc

_N, _H, _W = 32, 256, 256
_HW = _H * _W
_P = 8192
_CLS_W, _REG_W = 1.0, 1.2

_NC, _NS, _L = 2, 16, 16
_NWORK = _NC * _NS
_BPW = _P // _NWORK
_NCHUNK = _BPW // _L

_GRID = 2
_ROWS = _N * _H
_BLKR = _ROWS // _GRID


def _sc_body(pbi_hbm, pmi_hbm, boxes_hbm, gt_hbm, score_hbm, part_hbm,
             pbi_v, pmi_v, key_v, key4_v, pcol_v, scr_v, gt_v, acc_v, sem):
    wid = lax.axis_index("s") * _NC + lax.axis_index("c")
    base = wid * _BPW
    d0 = pltpu.async_copy(pbi_hbm.at[pl.ds(base, _BPW)], pbi_v, sem)
    d1 = pltpu.async_copy(pmi_hbm.at[pl.ds(base, _BPW)], pmi_v, sem)
    d2 = pltpu.async_copy(gt_hbm, gt_v, sem)
    d0.wait()
    d1.wait()
    d2.wait()

    for j in range(_NCHUNK):
        pb = pbi_v[pl.ds(j * _L, _L)]
        pm = pmi_v[pl.ds(j * _L, _L)]
        skey = (pb * _HW + ((pm >> 11) << 11) + ((pm >> 7) & 1) * 1024
                + (((pm >> 8) & 7) << 7) + (pm & 127))
        key_v[j // 8, pl.ds((j % 8) * _L, _L)] = skey
        base = pb * (_HW * 4) + ((pm >> 7) << 9) + (pm & 127)
        for c in range(4):
            key4_v[c, j // 8, pl.ds((j % 8) * _L, _L)] = base + (c << 7)

    descs = []
    for half in range(2):
        descs.append(pltpu.async_copy(score_hbm.at[key_v.at[half]],
                                      scr_v.at[half], sem))
        for c in range(4):
            descs.append(pltpu.async_copy(boxes_hbm.at[key4_v.at[c, half]],
                                          pcol_v.at[c, half], sem))
    for d in descs:
        d.wait()

    accd = jnp.zeros((_L,), jnp.float32)
    accr = jnp.zeros((_L,), jnp.float32)
    cols = [jnp.full((_L,), c, jnp.int32) for c in range(4)]
    for j in range(_NCHUNK):
        pb = pbi_v[pl.ds(j * _L, _L)]
        px1 = pcol_v[0, j // 8, pl.ds((j % 8) * _L, _L)]
        py1 = pcol_v[1, j // 8, pl.ds((j % 8) * _L, _L)]
        px2 = pcol_v[2, j // 8, pl.ds((j % 8) * _L, _L)]
        py2 = pcol_v[3, j // 8, pl.ds((j % 8) * _L, _L)]
        gx1 = plsc.load_gather(gt_v, [pb, cols[0]])
        gy1 = plsc.load_gather(gt_v, [pb, cols[1]])
        gx2 = plsc.load_gather(gt_v, [pb, cols[2]])
        gy2 = plsc.load_gather(gt_v, [pb, cols[3]])
        ww = jnp.maximum(jnp.minimum(gx2, px2) - jnp.maximum(gx1, px1), 0.0)
        hh = jnp.maximum(jnp.minimum(gy2, py2) - jnp.maximum(gy1, py1), 0.0)
        overlap = ww * hh
        a1 = (gx2 - gx1) * (gy2 - gy1)
        a2 = (px2 - px1) * (py2 - py1)
        union = jnp.maximum(a1 + a2 - overlap, 1e-6)
        iou = overlap / union
        sc = scr_v[j // 8, pl.ds((j % 8) * _L, _L)]
        accd = accd + sc * iou
        accr = accr + (jnp.abs(px1 - gx1) + jnp.abs(py1 - gy1)
                       + jnp.abs(px2 - gx2) + jnp.abs(py2 - gy2))

    acc_v[pl.ds(0, _L)] = accd
    acc_v[pl.ds(_L, _L)] = accr
    pltpu.sync_copy(acc_v, part_hbm.at[wid])


def _make_sc_call():
    return pl.kernel(
        _sc_body,
        out_type=jax.ShapeDtypeStruct((_NWORK, 2 * _L), jnp.float32),
        mesh=plsc.VectorSubcoreMesh(core_axis_name="c", subcore_axis_name="s",
                                    num_cores=_NC, num_subcores=_NS),
        compiler_params=pltpu.CompilerParams(needs_layout_passes=False),
        scratch_types=[
            pltpu.VMEM((_BPW,), jnp.int32),
            pltpu.VMEM((_BPW,), jnp.int32),
            pltpu.VMEM((2, 128), jnp.int32),
            pltpu.VMEM((4, 2, 128), jnp.int32),
            pltpu.VMEM((4, 2, 128), jnp.float32),
            pltpu.VMEM((2, 128), jnp.float32),
            pltpu.VMEM((_N, 4), jnp.float32),
            pltpu.VMEM((2 * _L,), jnp.float32),
            pltpu.SemaphoreType.DMA,
        ],
    )


def _tc_body(score_ref, fsum_ref, acc_ref):
    i = pl.program_id(0)

    @pl.when(i == 0)
    def _init():
        acc_ref[0, 0] = 0.0

    x = score_ref[...]
    f = jnp.maximum(x, 0.0) + jnp.log(1.0 + jnp.exp(-jnp.abs(x)))
    acc_ref[0, 0] += jnp.sum(f)

    @pl.when(i == _GRID - 1)
    def _fin():
        fsum_ref[0, 0] = acc_ref[0, 0]


_tc_call = pl.pallas_call(
    _tc_body,
    grid=(_GRID,),
    in_specs=[pl.BlockSpec((_BLKR, _W), lambda i: (i, 0))],
    out_specs=pl.BlockSpec(memory_space=pltpu.SMEM),
    out_shape=jax.ShapeDtypeStruct((1, 1), jnp.float32),
    scratch_shapes=[pltpu.SMEM((1, 1), jnp.float32)],
)


def _comb_body(np_ref, fsum_ref, part_ref, tot_ref, cls_ref, reg_ref):
    p = part_ref[...]
    dot = jnp.sum(p[:, :_L])
    regs = jnp.sum(p[:, _L:])
    npos = jnp.maximum(np_ref[0, 0], 1.0)
    cls = (fsum_ref[0, 0] - dot) / npos * _CLS_W
    reg = regs / npos * _REG_W
    tot_ref[0, 0] = cls + reg
    cls_ref[0, 0] = cls
    reg_ref[0, 0] = reg


_comb_call = pl.pallas_call(
    _comb_body,
    in_specs=[
        pl.BlockSpec(memory_space=pltpu.SMEM),
        pl.BlockSpec(memory_space=pltpu.SMEM),
        pl.BlockSpec((_NWORK, 2 * _L), lambda: (0, 0)),
    ],
    out_specs=[pl.BlockSpec(memory_space=pltpu.SMEM)] * 3,
    out_shape=[jax.ShapeDtypeStruct((1, 1), jnp.float32)] * 3,
)


def kernel(score_map, boxes, gt_boxes, num_positive_samples,
           positive_batch_idx, positive_map_idx):
    score_flat = (score_map.astype(jnp.float32)
                  .reshape(_N, _H // 8, 8, _W // 128, 128)
                  .transpose(0, 1, 3, 2, 4)
                  .reshape(-1))
    boxes_flat = (boxes.astype(jnp.float32)
                  .reshape(_N, _HW // 128, 128, 4)
                  .transpose(0, 1, 3, 2)
                  .reshape(-1))
    gt = gt_boxes.astype(jnp.float32)
    part = _make_sc_call()(positive_batch_idx, positive_map_idx, boxes_flat,
                           gt, score_flat)
    score2d = score_map.astype(jnp.float32).reshape(_ROWS, _W)
    np11 = jnp.reshape(num_positive_samples, (1, 1)).astype(jnp.float32)
    fsum = _tc_call(score2d)
    tot, cls, reg = _comb_call(np11, fsum, part)
    return (jnp.reshape(tot, ()), jnp.reshape(cls, ()), jnp.reshape(reg, ()))


# speedup vs baseline: 1.2485x; 1.0144x over previous
import jax
import jax.numpy as jnp
from jax import lax
from jax.experimental import pallas as pl
from jax.experimental.pallas import tpu as pltpu
from jax.experimental.pallas import tpu_sc as plsc

_N, _H, _W = 32, 256, 256
_HW = _H * _W
_P = 8192
_CLS_W, _REG_W = 1.0, 1.2

_NC, _NS, _L = 2, 16, 16
_NWORK = _NC * _NS
_BPW = _P // _NWORK
_NCHUNK = _BPW // _L

_GRID = 2
_ROWS = _N * _H
_BLKR = _ROWS // _GRID


def _sc_body(pbi_hbm, pmi_hbm, boxes_hbm, gt_hbm, score_hbm, part_hbm,
             pbi_v, pmi_v, key_v, key4_v, pcol_v, scr_v, gt_v, acc_v,
             sem_a, sem_b):
    wid = lax.axis_index("s") * _NC + lax.axis_index("c")
    base = wid * _BPW
    d0 = pltpu.async_copy(pbi_hbm.at[pl.ds(base, _BPW)], pbi_v, sem_a)
    d1 = pltpu.async_copy(pmi_hbm.at[pl.ds(base, _BPW)], pmi_v, sem_a)
    d2 = pltpu.async_copy(gt_hbm, gt_v, sem_a)
    d0.wait()
    d1.wait()
    d2.wait()

    halves = []
    for half in range(2):
        sem = sem_a if half == 0 else sem_b
        for j in range(8 * half, 8 * half + 8):
            pb = pbi_v[pl.ds(j * _L, _L)]
            pm = pmi_v[pl.ds(j * _L, _L)]
            skey = (pb * _HW + ((pm >> 11) << 11) + ((pm >> 7) & 1) * 1024
                    + (((pm >> 8) & 7) << 7) + (pm & 127))
            key_v[half, pl.ds((j % 8) * _L, _L)] = skey
            bkey = pb * (_HW * 4) + ((pm >> 7) << 9) + (pm & 127)
            for c in range(4):
                key4_v[c, half, pl.ds((j % 8) * _L, _L)] = bkey + (c << 7)
        descs = [pltpu.async_copy(score_hbm.at[key_v.at[half]],
                                  scr_v.at[half], sem)]
        for c in range(4):
            descs.append(pltpu.async_copy(boxes_hbm.at[key4_v.at[c, half]],
                                          pcol_v.at[c, half], sem))
        halves.append(descs)

    accd = jnp.zeros((_L,), jnp.float32)
    accr = jnp.zeros((_L,), jnp.float32)
    cols = [jnp.full((_L,), c, jnp.int32) for c in range(4)]
    for half in range(2):
        for d in halves[half]:
            d.wait()
        for j in range(8 * half, 8 * half + 8):
            pb = pbi_v[pl.ds(j * _L, _L)]
            px1 = pcol_v[0, half, pl.ds((j % 8) * _L, _L)]
            py1 = pcol_v[1, half, pl.ds((j % 8) * _L, _L)]
            px2 = pcol_v[2, half, pl.ds((j % 8) * _L, _L)]
            py2 = pcol_v[3, half, pl.ds((j % 8) * _L, _L)]
            gx1 = plsc.load_gather(gt_v, [pb, cols[0]])
            gy1 = plsc.load_gather(gt_v, [pb, cols[1]])
            gx2 = plsc.load_gather(gt_v, [pb, cols[2]])
            gy2 = plsc.load_gather(gt_v, [pb, cols[3]])
            ww = jnp.maximum(jnp.minimum(gx2, px2) - jnp.maximum(gx1, px1),
                             0.0)
            hh = jnp.maximum(jnp.minimum(gy2, py2) - jnp.maximum(gy1, py1),
                             0.0)
            overlap = ww * hh
            a1 = (gx2 - gx1) * (gy2 - gy1)
            a2 = (px2 - px1) * (py2 - py1)
            union = jnp.maximum(a1 + a2 - overlap, 1e-6)
            iou = overlap / union
            sc = scr_v[half, pl.ds((j % 8) * _L, _L)]
            accd = accd + sc * iou
            accr = accr + (jnp.abs(px1 - gx1) + jnp.abs(py1 - gy1)
                           + jnp.abs(px2 - gx2) + jnp.abs(py2 - gy2))

    acc_v[pl.ds(0, _L)] = accd
    acc_v[pl.ds(_L, _L)] = accr
    pltpu.sync_copy(acc_v, part_hbm.at[wid])


def _make_sc_call():
    return pl.kernel(
        _sc_body,
        out_type=jax.ShapeDtypeStruct((_NWORK, 2 * _L), jnp.float32),
        mesh=plsc.VectorSubcoreMesh(core_axis_name="c", subcore_axis_name="s",
                                    num_cores=_NC, num_subcores=_NS),
        compiler_params=pltpu.CompilerParams(needs_layout_passes=False),
        scratch_types=[
            pltpu.VMEM((_BPW,), jnp.int32),
            pltpu.VMEM((_BPW,), jnp.int32),
            pltpu.VMEM((2, 128), jnp.int32),
            pltpu.VMEM((4, 2, 128), jnp.int32),
            pltpu.VMEM((4, 2, 128), jnp.float32),
            pltpu.VMEM((2, 128), jnp.float32),
            pltpu.VMEM((_N, 4), jnp.float32),
            pltpu.VMEM((2 * _L,), jnp.float32),
            pltpu.SemaphoreType.DMA,
            pltpu.SemaphoreType.DMA,
        ],
    )


def _tc_body(score_ref, fsum_ref, acc_ref):
    i = pl.program_id(0)

    @pl.when(i == 0)
    def _init():
        acc_ref[0, 0] = 0.0

    x = score_ref[...]
    f = jnp.maximum(x, 0.0) + jnp.log(1.0 + jnp.exp(-jnp.abs(x)))
    acc_ref[0, 0] += jnp.sum(f)

    @pl.when(i == _GRID - 1)
    def _fin():
        fsum_ref[0, 0] = acc_ref[0, 0]


_tc_call = pl.pallas_call(
    _tc_body,
    grid=(_GRID,),
    in_specs=[pl.BlockSpec((_BLKR, _W), lambda i: (i, 0))],
    out_specs=pl.BlockSpec(memory_space=pltpu.SMEM),
    out_shape=jax.ShapeDtypeStruct((1, 1), jnp.float32),
    scratch_shapes=[pltpu.SMEM((1, 1), jnp.float32)],
)


def _comb_body(np_ref, fsum_ref, part_ref, tot_ref, cls_ref, reg_ref):
    p = part_ref[...]
    dot = jnp.sum(p[:, :_L])
    regs = jnp.sum(p[:, _L:])
    npos = jnp.maximum(np_ref[0, 0], 1.0)
    cls = (fsum_ref[0, 0] - dot) / npos * _CLS_W
    reg = regs / npos * _REG_W
    tot_ref[0, 0] = cls + reg
    cls_ref[0, 0] = cls
    reg_ref[0, 0] = reg


_comb_call = pl.pallas_call(
    _comb_body,
    in_specs=[
        pl.BlockSpec(memory_space=pltpu.SMEM),
        pl.BlockSpec(memory_space=pltpu.SMEM),
        pl.BlockSpec((_NWORK, 2 * _L), lambda: (0, 0)),
    ],
    out_specs=[pl.BlockSpec(memory_space=pltpu.SMEM)] * 3,
    out_shape=[jax.ShapeDtypeStruct((1, 1), jnp.float32)] * 3,
)


def kernel(score_map, boxes, gt_boxes, num_positive_samples,
           positive_batch_idx, positive_map_idx):
    score_flat = (score_map.astype(jnp.float32)
                  .reshape(_N, _H // 8, 8, _W // 128, 128)
                  .transpose(0, 1, 3, 2, 4)
                  .reshape(-1))
    boxes_flat = (boxes.astype(jnp.float32)
                  .reshape(_N, _HW // 128, 128, 4)
                  .transpose(0, 1, 3, 2)
                  .reshape(-1))
    gt = gt_boxes.astype(jnp.float32)
    part = _make_sc_call()(positive_batch_idx, positive_map_idx, boxes_flat,
                           gt, score_flat)
    score2d = score_map.astype(jnp.float32).reshape(_ROWS, _W)
    np11 = jnp.reshape(num_positive_samples, (1, 1)).astype(jnp.float32)
    fsum = _tc_call(score2d)
    tot, cls, reg = _comb_call(np11, fsum, part)
    return (jnp.reshape(tot, ()), jnp.reshape(cls, ()), jnp.reshape(reg, ()))
